# Initial kernel scaffold; baseline (speedup 1.0000x reference)
import jax
import jax.numpy as jnp
from jax.experimental import pallas as pl


def kernel(x, edge_index, W_self0, W_neigh0, b0, W_self1, W_neigh1, b1, W_self2, W_neigh2, b2):
    raise NotImplementedError("write your pallas kernel here")


# speedup vs baseline: 2.1618x; 2.1618x over previous
import functools

import jax
import jax.numpy as jnp
from jax import lax
from jax.experimental import pallas as pl
from jax.experimental.pallas import tpu as pltpu
from jax.experimental.pallas import tpu_sc as plsc

N = 10000
E = 160000
D_IN = 256
D_H = 512
D_OUT = 128

NPAD = 10240
EPAD = 163840
NC = 2
NS = 16
EPT = EPAD // (NC * NS)
KB = 128
NBLK = EPT // KB
ROWS_PT = NPAD // NS
BN = 256
GRID = NPAD // BN


def _sc_mesh():
    return plsc.VectorSubcoreMesh(core_axis_name="c", subcore_axis_name="s")


@functools.partial(
    pl.kernel,
    out_type=jax.ShapeDtypeStruct((NC, NPAD, 128), jnp.float32),
    mesh=_sc_mesh(),
    scratch_types=[
        pltpu.VMEM((KB,), jnp.int32),
        pltpu.VMEM((KB,), jnp.int32),
        pltpu.VMEM((KB, 128), jnp.float32),
        pltpu.VMEM_SHARED((NPAD, 128), jnp.float32),
        pltpu.SemaphoreType.DMA,
    ],
)
def _sc_agg_chunk(table, src, dst, zeros, out, src_v, dst_v, rows_v, accum, sem):
    cid = lax.axis_index("c")
    sid = lax.axis_index("s")
    wid = cid * NS + sid

    row0 = sid * ROWS_PT
    pltpu.sync_copy(zeros.at[pl.ds(row0, ROWS_PT)], accum.at[pl.ds(row0, ROWS_PT)])
    plsc.subcore_barrier()

    ebase = wid * EPT

    def body(i, carry):
        base = pl.multiple_of(ebase + i * KB, KB)
        pltpu.sync_copy(src.at[pl.ds(base, KB)], src_v)
        pltpu.sync_copy(dst.at[pl.ds(base, KB)], dst_v)
        pltpu.async_copy(table.at[src_v], rows_v, sem).wait()
        pltpu.sync_copy(rows_v, accum.at[dst_v], add=True)
        return carry

    lax.fori_loop(0, NBLK, body, 0)
    plsc.subcore_barrier()

    pltpu.sync_copy(accum.at[pl.ds(row0, ROWS_PT)], out.at[cid, pl.ds(row0, ROWS_PT)])


@functools.partial(
    pl.kernel,
    out_type=jax.ShapeDtypeStruct((NC, NPAD, 128), jnp.float32),
    mesh=_sc_mesh(),
    scratch_types=[
        pltpu.VMEM((KB,), jnp.int32),
        pltpu.VMEM((KB, 128), jnp.float32),
        pltpu.VMEM_SHARED((NPAD, 128), jnp.float32),
        pltpu.SemaphoreType.DMA,
    ],
)
def _sc_degree(dst, zeros, out, dst_v, ones_v, accum, sem):
    cid = lax.axis_index("c")
    sid = lax.axis_index("s")
    wid = cid * NS + sid

    row0 = sid * ROWS_PT
    pltpu.sync_copy(zeros.at[pl.ds(row0, ROWS_PT)], accum.at[pl.ds(row0, ROWS_PT)])

    ones16 = jnp.ones((16,), jnp.float32)

    def obody(i, carry):
        for j in range(128 // 16):
            ones_v[i, pl.ds(j * 16, 16)] = ones16
        return carry

    lax.fori_loop(0, KB, obody, 0)
    plsc.subcore_barrier()

    ebase = wid * EPT

    def body(i, carry):
        base = pl.multiple_of(ebase + i * KB, KB)
        pltpu.sync_copy(dst.at[pl.ds(base, KB)], dst_v)
        pltpu.sync_copy(ones_v, accum.at[dst_v], add=True)
        return carry

    lax.fori_loop(0, NBLK, body, 0)
    plsc.subcore_barrier()

    pltpu.sync_copy(accum.at[pl.ds(row0, ROWS_PT)], out.at[cid, pl.ds(row0, ROWS_PT)])


def _make_mm_layer(cin, dout, relu):
    cout = dout // 128
    din = cin * 128

    def body(h_ref, *rest):
        agg_refs = rest[:cin]
        deg_ref, ws_ref, wn_ref, b_ref, out_ref = rest[cin:]
        d = deg_ref[0] + deg_ref[1]
        r = 1.0 / jnp.maximum(d, 1.0)
        acc = jnp.zeros((BN, dout), jnp.float32)
        for c in range(cin):
            hb = h_ref[c]
            ub = (agg_refs[c][0] + agg_refs[c][1]) * r
            ws = ws_ref[pl.ds(c * 128, 128), :]
            wn = wn_ref[pl.ds(c * 128, 128), :]
            acc = acc + jnp.dot(hb, ws, preferred_element_type=jnp.float32)
            acc = acc + jnp.dot(ub, wn, preferred_element_type=jnp.float32)
        acc = acc + b_ref[0]
        if relu:
            acc = jnp.maximum(acc, 0.0)
        for co in range(cout):
            out_ref[co] = acc[:, co * 128:(co + 1) * 128]

    in_specs = (
        [pl.BlockSpec((cin, BN, 128), lambda i: (0, i, 0))]
        + [pl.BlockSpec((NC, BN, 128), lambda i: (0, i, 0)) for _ in range(cin)]
        + [
            pl.BlockSpec((NC, BN, 1), lambda i: (0, i, 0)),
            pl.BlockSpec((din, dout), lambda i: (0, 0)),
            pl.BlockSpec((din, dout), lambda i: (0, 0)),
            pl.BlockSpec((1, dout), lambda i: (0, 0)),
        ]
    )

    return pl.pallas_call(
        body,
        grid=(GRID,),
        in_specs=in_specs,
        out_specs=pl.BlockSpec((cout, BN, 128), lambda i: (0, i, 0)),
        out_shape=jax.ShapeDtypeStruct((cout, NPAD, 128), jnp.float32),
    )


def _mm_layer2_body(h_ref, ws_ref, wn_ref, b_ref, s_ref, t_ref):
    s = jnp.zeros((BN, D_OUT), jnp.float32)
    t = jnp.zeros((BN, D_OUT), jnp.float32)
    for c in range(D_H // 128):
        hb = h_ref[c]
        s = s + jnp.dot(hb, ws_ref[pl.ds(c * 128, 128), :],
                        preferred_element_type=jnp.float32)
        t = t + jnp.dot(hb, wn_ref[pl.ds(c * 128, 128), :],
                        preferred_element_type=jnp.float32)
    s_ref[...] = s + b_ref[0]
    t_ref[...] = t


_mm_layer2 = pl.pallas_call(
    _mm_layer2_body,
    grid=(GRID,),
    in_specs=[
        pl.BlockSpec((D_H // 128, BN, 128), lambda i: (0, i, 0)),
        pl.BlockSpec((D_H, D_OUT), lambda i: (0, 0)),
        pl.BlockSpec((D_H, D_OUT), lambda i: (0, 0)),
        pl.BlockSpec((1, D_OUT), lambda i: (0, 0)),
    ],
    out_specs=[
        pl.BlockSpec((BN, D_OUT), lambda i: (i, 0)),
        pl.BlockSpec((BN, D_OUT), lambda i: (i, 0)),
    ],
    out_shape=[
        jax.ShapeDtypeStruct((NPAD, D_OUT), jnp.float32),
        jax.ShapeDtypeStruct((NPAD, D_OUT), jnp.float32),
    ],
)


def _final_body(s_ref, agg_ref, deg_ref, out_ref):
    d = deg_ref[0] + deg_ref[1]
    r = 1.0 / jnp.maximum(d, 1.0)
    out_ref[...] = s_ref[...] + (agg_ref[0] + agg_ref[1]) * r


_final_add = pl.pallas_call(
    _final_body,
    grid=(GRID,),
    in_specs=[
        pl.BlockSpec((BN, D_OUT), lambda i: (i, 0)),
        pl.BlockSpec((NC, BN, 128), lambda i: (0, i, 0)),
        pl.BlockSpec((NC, BN, 1), lambda i: (0, i, 0)),
    ],
    out_specs=pl.BlockSpec((BN, D_OUT), lambda i: (i, 0)),
    out_shape=jax.ShapeDtypeStruct((NPAD, D_OUT), jnp.float32),
)


def kernel(x, edge_index, W_self0, W_neigh0, b0, W_self1, W_neigh1, b1,
           W_self2, W_neigh2, b2):
    f32 = jnp.float32
    src = edge_index[0]
    dst = edge_index[1]
    pad_e = EPAD - E
    src_p = jnp.concatenate([src, jnp.zeros((pad_e,), jnp.int32)])
    dst_p = jnp.concatenate([dst, jnp.full((pad_e,), N, jnp.int32)])

    xp = jnp.pad(x, ((0, NPAD - N), (0, 0)))
    x_c = xp.reshape(NPAD, D_IN // 128, 128).transpose(1, 0, 2)

    zeros_slab = jnp.zeros((NPAD, 128), f32)

    degp = _sc_degree(dst_p, zeros_slab)
    deg3 = degp[:, :, :1]

    b0r = b0.reshape(1, D_H)
    b1r = b1.reshape(1, D_H)
    b2r = b2.reshape(1, D_OUT)

    aggs0 = [_sc_agg_chunk(x_c[c], src_p, dst_p, zeros_slab)
             for c in range(D_IN // 128)]
    h1 = _make_mm_layer(D_IN // 128, D_H, True)(
        x_c, *aggs0, deg3, W_self0, W_neigh0, b0r)

    aggs1 = [_sc_agg_chunk(h1[c], src_p, dst_p, zeros_slab)
             for c in range(D_H // 128)]
    h2 = _make_mm_layer(D_H // 128, D_H, True)(
        h1, *aggs1, deg3, W_self1, W_neigh1, b1r)

    s, t = _mm_layer2(h2, W_self2, W_neigh2, b2r)
    agg2 = _sc_agg_chunk(t, src_p, dst_p, zeros_slab)
    out = _final_add(s, agg2, deg3)

    return out[:N]


# speedup vs baseline: 2.6604x; 1.2306x over previous
import functools

import jax
import jax.numpy as jnp
from jax import lax
from jax.experimental import pallas as pl
from jax.experimental.pallas import tpu as pltpu
from jax.experimental.pallas import tpu_sc as plsc

N = 10000
E = 160000
D_IN = 256
D_H = 512
D_OUT = 128

NPAD = 10240
EPAD = 163840
NC = 2
NS = 16
EPT = EPAD // (NC * NS)
KB = 128
NBLK = EPT // KB
ROWS_PT = NPAD // NS
BN = 256
GRID = NPAD // BN


def _sc_mesh():
    return plsc.VectorSubcoreMesh(core_axis_name="c", subcore_axis_name="s")


NBUF = 2
NBI = 4


@functools.partial(
    pl.kernel,
    out_type=jax.ShapeDtypeStruct((NC, NPAD, 128), jnp.float32),
    mesh=_sc_mesh(),
    scratch_types=[
        pltpu.VMEM((NBI, KB), jnp.int32),
        pltpu.VMEM((NBI, KB), jnp.int32),
        pltpu.VMEM((NBUF, KB, 128), jnp.float32),
        pltpu.VMEM_SHARED((NPAD, 128), jnp.float32),
        pltpu.SemaphoreType.DMA((NBI,)),
        pltpu.SemaphoreType.DMA((NBUF,)),
        pltpu.SemaphoreType.DMA((NBUF,)),
    ],
)
def _sc_agg_chunk(table, src, dst, zeros, out, src_v, dst_v, rows_v, accum,
                  sem_i, sem_g, sem_s):
    cid = lax.axis_index("c")
    sid = lax.axis_index("s")
    wid = cid * NS + sid

    row0 = sid * ROWS_PT
    pltpu.sync_copy(zeros.at[pl.ds(row0, ROWS_PT)], accum.at[pl.ds(row0, ROWS_PT)])
    plsc.subcore_barrier()

    ebase = wid * EPT

    def bbase(j):
        return pl.multiple_of(ebase + j * KB, KB)

    def issue_idx(j, r):
        base = bbase(j)
        pltpu.async_copy(src.at[pl.ds(base, KB)], src_v.at[r], sem_i.at[r])
        pltpu.async_copy(dst.at[pl.ds(base, KB)], dst_v.at[r], sem_i.at[r])

    def wait_idx(j, r):
        base = bbase(j)
        pltpu.make_async_copy(
            src.at[pl.ds(base, KB)], src_v.at[r], sem_i.at[r]).wait()
        pltpu.make_async_copy(
            dst.at[pl.ds(base, KB)], dst_v.at[r], sem_i.at[r]).wait()

    def wait_scat(r):
        pltpu.make_async_copy(
            rows_v.at[r % NBUF], accum.at[dst_v.at[r % NBI]],
            sem_s.at[r % NBUF]).wait()

    def issue_scat(r):
        pltpu.async_copy(rows_v.at[r % NBUF], accum.at[dst_v.at[r % NBI]],
                         sem_s.at[r % NBUF], add=True)

    def issue_gather(r):
        pltpu.async_copy(table.at[src_v.at[r % NBI]], rows_v.at[r % NBUF],
                         sem_g.at[r % NBUF])

    def wait_gather(r):
        pltpu.make_async_copy(
            table.at[src_v.at[r % NBI]], rows_v.at[r % NBUF],
            sem_g.at[r % NBUF]).wait()

    def block(j, r, s1=True, s2=True, s3=True, s4=True):
        if s1:
            wait_scat((r - 2) % NBI)
        if s2:
            issue_idx(j + 1, (r + 1) % NBI)
        if s3:
            wait_idx(j, r)
            issue_gather(r)
        if s4:
            wait_gather((r - 1) % NBI)
            issue_scat((r - 1) % NBI)

    issue_idx(0, 0)
    block(0, 0, s1=False, s4=False)
    block(1, 1, s1=False)

    def gbody(g, carry):
        j0 = g * NBI + 2
        for u in range(NBI):
            block(j0 + u, (2 + u) % NBI)
        return carry

    lax.fori_loop(0, (NBLK - 4) // NBI, gbody, 0)

    block(NBLK - 2, (NBLK - 2) % NBI)
    block(NBLK - 1, (NBLK - 1) % NBI, s2=False)
    wait_gather((NBLK - 1) % NBI)
    issue_scat((NBLK - 1) % NBI)
    wait_scat((NBLK - 2) % NBI)
    wait_scat((NBLK - 1) % NBI)
    plsc.subcore_barrier()

    pltpu.sync_copy(accum.at[pl.ds(row0, ROWS_PT)], out.at[cid, pl.ds(row0, ROWS_PT)])


DEG_W = 128


@functools.partial(
    pl.kernel,
    out_type=jax.ShapeDtypeStruct((NC, NPAD, DEG_W), jnp.float32),
    mesh=_sc_mesh(),
    scratch_types=[
        pltpu.VMEM((NBI, KB), jnp.int32),
        pltpu.VMEM((KB, DEG_W), jnp.float32),
        pltpu.VMEM_SHARED((NPAD, DEG_W), jnp.float32),
        pltpu.SemaphoreType.DMA((NBI,)),
        pltpu.SemaphoreType.DMA((NBUF,)),
    ],
)
def _sc_degree(dst, zeros, out, dst_v, ones_v, accum, sem_i, sem_s):
    cid = lax.axis_index("c")
    sid = lax.axis_index("s")
    wid = cid * NS + sid

    row0 = sid * ROWS_PT
    pltpu.sync_copy(zeros.at[pl.ds(row0, ROWS_PT)], accum.at[pl.ds(row0, ROWS_PT)])

    ones16 = jnp.ones((DEG_W,), jnp.float32)

    def obody(i, carry):
        ones_v[i, pl.ds(0, DEG_W)] = ones16
        return carry

    lax.fori_loop(0, KB, obody, 0)
    plsc.subcore_barrier()

    ebase = wid * EPT

    def bbase(j):
        return pl.multiple_of(ebase + j * KB, KB)

    def issue_idx(j, r):
        pltpu.async_copy(dst.at[pl.ds(bbase(j), KB)], dst_v.at[r],
                         sem_i.at[r])

    def wait_idx(j, r):
        pltpu.make_async_copy(
            dst.at[pl.ds(bbase(j), KB)], dst_v.at[r], sem_i.at[r]).wait()

    def wait_scat(r):
        pltpu.make_async_copy(
            ones_v, accum.at[dst_v.at[r % NBI]], sem_s.at[r % NBUF]).wait()

    def block(j, r, t1=True, t2=True):
        if t1:
            wait_scat((r - 2) % NBI)
        if t2:
            issue_idx(j + 1, (r + 1) % NBI)
        wait_idx(j, r)
        pltpu.async_copy(ones_v, accum.at[dst_v.at[r]], sem_s.at[r % NBUF],
                         add=True)

    issue_idx(0, 0)
    block(0, 0, t1=False)
    block(1, 1, t1=False)

    def gbody(g, carry):
        j0 = g * NBI + 2
        for u in range(NBI):
            block(j0 + u, (2 + u) % NBI)
        return carry

    lax.fori_loop(0, (NBLK - 4) // NBI, gbody, 0)

    block(NBLK - 2, (NBLK - 2) % NBI)
    block(NBLK - 1, (NBLK - 1) % NBI, t2=False)
    wait_scat((NBLK - 2) % NBI)
    wait_scat((NBLK - 1) % NBI)
    plsc.subcore_barrier()

    pltpu.sync_copy(accum.at[pl.ds(row0, ROWS_PT)], out.at[cid, pl.ds(row0, ROWS_PT)])


def _make_mm_layer(cin, dout, relu):
    cout = dout // 128
    din = cin * 128

    def body(h_ref, *rest):
        agg_refs = rest[:cin]
        deg_ref, ws_ref, wn_ref, b_ref, out_ref = rest[cin:]
        d = deg_ref[0] + deg_ref[1]
        r = 1.0 / jnp.maximum(d, 1.0)
        acc = jnp.zeros((BN, dout), jnp.float32)
        for c in range(cin):
            hb = h_ref[c]
            ub = (agg_refs[c][0] + agg_refs[c][1]) * r
            ws = ws_ref[pl.ds(c * 128, 128), :]
            wn = wn_ref[pl.ds(c * 128, 128), :]
            acc = acc + jnp.dot(hb, ws, preferred_element_type=jnp.float32)
            acc = acc + jnp.dot(ub, wn, preferred_element_type=jnp.float32)
        acc = acc + b_ref[0]
        if relu:
            acc = jnp.maximum(acc, 0.0)
        for co in range(cout):
            out_ref[co] = acc[:, co * 128:(co + 1) * 128]

    in_specs = (
        [pl.BlockSpec((cin, BN, 128), lambda i: (0, i, 0))]
        + [pl.BlockSpec((NC, BN, 128), lambda i: (0, i, 0)) for _ in range(cin)]
        + [
            pl.BlockSpec((NC, BN, 1), lambda i: (0, i, 0)),
            pl.BlockSpec((din, dout), lambda i: (0, 0)),
            pl.BlockSpec((din, dout), lambda i: (0, 0)),
            pl.BlockSpec((1, dout), lambda i: (0, 0)),
        ]
    )

    return pl.pallas_call(
        body,
        grid=(GRID,),
        in_specs=in_specs,
        out_specs=pl.BlockSpec((cout, BN, 128), lambda i: (0, i, 0)),
        out_shape=jax.ShapeDtypeStruct((cout, NPAD, 128), jnp.float32),
    )


def _mm_layer2_body(h_ref, ws_ref, wn_ref, b_ref, s_ref, t_ref):
    s = jnp.zeros((BN, D_OUT), jnp.float32)
    t = jnp.zeros((BN, D_OUT), jnp.float32)
    for c in range(D_H // 128):
        hb = h_ref[c]
        s = s + jnp.dot(hb, ws_ref[pl.ds(c * 128, 128), :],
                        preferred_element_type=jnp.float32)
        t = t + jnp.dot(hb, wn_ref[pl.ds(c * 128, 128), :],
                        preferred_element_type=jnp.float32)
    s_ref[...] = s + b_ref[0]
    t_ref[...] = t


_mm_layer2 = pl.pallas_call(
    _mm_layer2_body,
    grid=(GRID,),
    in_specs=[
        pl.BlockSpec((D_H // 128, BN, 128), lambda i: (0, i, 0)),
        pl.BlockSpec((D_H, D_OUT), lambda i: (0, 0)),
        pl.BlockSpec((D_H, D_OUT), lambda i: (0, 0)),
        pl.BlockSpec((1, D_OUT), lambda i: (0, 0)),
    ],
    out_specs=[
        pl.BlockSpec((BN, D_OUT), lambda i: (i, 0)),
        pl.BlockSpec((BN, D_OUT), lambda i: (i, 0)),
    ],
    out_shape=[
        jax.ShapeDtypeStruct((NPAD, D_OUT), jnp.float32),
        jax.ShapeDtypeStruct((NPAD, D_OUT), jnp.float32),
    ],
)


def _final_body(s_ref, agg_ref, deg_ref, out_ref):
    d = deg_ref[0] + deg_ref[1]
    r = 1.0 / jnp.maximum(d, 1.0)
    out_ref[...] = s_ref[...] + (agg_ref[0] + agg_ref[1]) * r


_final_add = pl.pallas_call(
    _final_body,
    grid=(GRID,),
    in_specs=[
        pl.BlockSpec((BN, D_OUT), lambda i: (i, 0)),
        pl.BlockSpec((NC, BN, 128), lambda i: (0, i, 0)),
        pl.BlockSpec((NC, BN, 1), lambda i: (0, i, 0)),
    ],
    out_specs=pl.BlockSpec((BN, D_OUT), lambda i: (i, 0)),
    out_shape=jax.ShapeDtypeStruct((NPAD, D_OUT), jnp.float32),
)


def kernel(x, edge_index, W_self0, W_neigh0, b0, W_self1, W_neigh1, b1,
           W_self2, W_neigh2, b2):
    f32 = jnp.float32
    src = edge_index[0]
    dst = edge_index[1]
    pad_e = EPAD - E
    src_p = jnp.concatenate([src, jnp.zeros((pad_e,), jnp.int32)])
    dst_p = jnp.concatenate([dst, jnp.full((pad_e,), N, jnp.int32)])

    xp = jnp.pad(x, ((0, NPAD - N), (0, 0)))
    x_c = xp.reshape(NPAD, D_IN // 128, 128).transpose(1, 0, 2)

    zeros_slab = jnp.zeros((NPAD, 128), f32)
    zeros_deg = jnp.zeros((NPAD, DEG_W), f32)

    degp = _sc_degree(dst_p, zeros_deg)
    deg3 = degp[:, :, :1]

    b0r = b0.reshape(1, D_H)
    b1r = b1.reshape(1, D_H)
    b2r = b2.reshape(1, D_OUT)

    aggs0 = [_sc_agg_chunk(x_c[c], src_p, dst_p, zeros_slab)
             for c in range(D_IN // 128)]
    h1 = _make_mm_layer(D_IN // 128, D_H, True)(
        x_c, *aggs0, deg3, W_self0, W_neigh0, b0r)

    aggs1 = [_sc_agg_chunk(h1[c], src_p, dst_p, zeros_slab)
             for c in range(D_H // 128)]
    h2 = _make_mm_layer(D_H // 128, D_H, True)(
        h1, *aggs1, deg3, W_self1, W_neigh1, b1r)

    s, t = _mm_layer2(h2, W_self2, W_neigh2, b2r)
    agg2 = _sc_agg_chunk(t, src_p, dst_p, zeros_slab)
    out = _final_add(s, agg2, deg3)

    return out[:N]
